# Initial kernel scaffold; baseline (speedup 1.0000x reference)
import jax
import jax.numpy as jnp
from jax.experimental import pallas as pl


def kernel(features, edge_index, W1, b1, W2, b2):
    raise NotImplementedError("write your pallas kernel here")


# speedup vs baseline: 3.5222x; 3.5222x over previous
import functools

import jax
import jax.numpy as jnp
from jax import lax
from jax.experimental import pallas as pl
from jax.experimental.pallas import tpu as pltpu
from jax.experimental.pallas import tpu_sc as plsc

N = 10000
E = 160000
D_IN = 256
H_FEATS = 512
N_CLASSES = 64
K = 10
ALPHA = 0.1

NC = 2
NS = 16
LANES = 16
NPAD = 10240
EPT = 5120
EPAD = NC * NS * EPT
CHUNK = 128
NCHUNK = EPT // CHUNK
ROWS_PER_TILE = NPAD // NS
_MESH = plsc.VectorSubcoreMesh(core_axis_name="c", subcore_axis_name="s")

_BN = 1000


def _mlp_body(x_ref, w1_ref, b1_ref, w2_ref, b2_ref, o_ref):
    h = jnp.dot(x_ref[...], w1_ref[...], preferred_element_type=jnp.float32)
    h = jnp.maximum(h + b1_ref[...], 0.0)
    o_ref[...] = jnp.dot(h, w2_ref[...], preferred_element_type=jnp.float32) + b2_ref[...]


def _mlp(x, w1, b1, w2, b2):
    return pl.pallas_call(
        _mlp_body,
        grid=(N // _BN,),
        in_specs=[
            pl.BlockSpec((_BN, D_IN), lambda i: (i, 0)),
            pl.BlockSpec((D_IN, H_FEATS), lambda i: (0, 0)),
            pl.BlockSpec((1, H_FEATS), lambda i: (0, 0)),
            pl.BlockSpec((H_FEATS, N_CLASSES), lambda i: (0, 0)),
            pl.BlockSpec((1, N_CLASSES), lambda i: (0, 0)),
        ],
        out_specs=pl.BlockSpec((_BN, N_CLASSES), lambda i: (i, 0)),
        out_shape=jax.ShapeDtypeStruct((N, N_CLASSES), jnp.float32),
    )(x, w1, b1, w2, b2)


_SC_PARAMS = pltpu.CompilerParams(use_tc_tiling_on_sc=False)


@functools.partial(
    pl.kernel,
    out_type=jax.ShapeDtypeStruct((NC, 2, NPAD, LANES), jnp.float32),
    mesh=_MESH,
    compiler_params=_SC_PARAMS,
    scratch_types=[
        pltpu.VMEM((CHUNK,), jnp.int32),
        pltpu.VMEM((CHUNK, LANES), jnp.float32),
        pltpu.VMEM((CHUNK, LANES), jnp.float32),
        pltpu.VMEM_SHARED((NPAD, LANES), jnp.float32),
        pltpu.VMEM_SHARED((NPAD, LANES), jnp.float32),
    ],
)
def _sc_degrees(src_hbm, dst_hbm, out_hbm, idx_v, ones_v, zero_v, dsrc_sh, ddst_sh):
    cid = lax.axis_index("c")
    sid = lax.axis_index("s")

    @pl.loop(0, CHUNK)
    def _(r):
        ones_v[r, :] = jnp.full((LANES,), 1.0, jnp.float32)
        zero_v[r, :] = jnp.zeros((LANES,), jnp.float32)

    base = sid * ROWS_PER_TILE
    for b in range(ROWS_PER_TILE // CHUNK):
        pltpu.sync_copy(zero_v, dsrc_sh.at[pl.ds(base + b * CHUNK, CHUNK)])
        pltpu.sync_copy(zero_v, ddst_sh.at[pl.ds(base + b * CHUNK, CHUNK)])
    plsc.subcore_barrier()

    ebase = (cid * NS + sid) * EPT

    @pl.loop(0, NCHUNK)
    def _(ch):
        off = ebase + ch * CHUNK
        pltpu.sync_copy(src_hbm.at[pl.ds(off, CHUNK)], idx_v)
        pltpu.sync_copy(ones_v, dsrc_sh.at[idx_v], add=True)
        pltpu.sync_copy(dst_hbm.at[pl.ds(off, CHUNK)], idx_v)
        pltpu.sync_copy(ones_v, ddst_sh.at[idx_v], add=True)

    plsc.subcore_barrier()
    pltpu.sync_copy(dsrc_sh.at[pl.ds(base, ROWS_PER_TILE)],
                    out_hbm.at[cid, 0, pl.ds(base, ROWS_PER_TILE)])
    pltpu.sync_copy(ddst_sh.at[pl.ds(base, ROWS_PER_TILE)],
                    out_hbm.at[cid, 1, pl.ds(base, ROWS_PER_TILE)])


@functools.partial(
    pl.kernel,
    out_type=jax.ShapeDtypeStruct((NC, NPAD, N_CLASSES), jnp.float32),
    mesh=_MESH,
    compiler_params=_SC_PARAMS,
    scratch_types=[
        pltpu.VMEM((CHUNK,), jnp.int32),
        pltpu.VMEM((CHUNK,), jnp.int32),
        pltpu.VMEM((CHUNK, N_CLASSES), jnp.float32),
        pltpu.VMEM((CHUNK, N_CLASSES), jnp.float32),
        pltpu.VMEM_SHARED((NPAD, N_CLASSES), jnp.float32),
        pltpu.SemaphoreType.DMA,
    ],
)
def _sc_step(hs_hbm, src_hbm, dst_hbm, out_hbm, srcv, dstv, rows_v, zero_v, agg_sh, sem):
    cid = lax.axis_index("c")
    sid = lax.axis_index("s")

    @pl.loop(0, CHUNK)
    def _(r):
        @pl.loop(0, N_CLASSES, step=LANES)
        def _(j):
            zero_v[r, pl.ds(j, LANES)] = jnp.zeros((LANES,), jnp.float32)

    base = sid * ROWS_PER_TILE
    for b in range(ROWS_PER_TILE // CHUNK):
        pltpu.sync_copy(zero_v, agg_sh.at[pl.ds(base + b * CHUNK, CHUNK)])
    plsc.subcore_barrier()

    ebase = (cid * NS + sid) * EPT

    @pl.loop(0, NCHUNK)
    def _(ch):
        off = ebase + ch * CHUNK
        pltpu.sync_copy(src_hbm.at[pl.ds(off, CHUNK)], srcv)
        pltpu.sync_copy(dst_hbm.at[pl.ds(off, CHUNK)], dstv)
        pltpu.async_copy(hs_hbm.at[srcv], rows_v, sem).wait()
        pltpu.sync_copy(rows_v, agg_sh.at[dstv], add=True)

    plsc.subcore_barrier()
    pltpu.sync_copy(agg_sh.at[pl.ds(base, ROWS_PER_TILE)],
                    out_hbm.at[cid, pl.ds(base, ROWS_PER_TILE)])


def _norm_body(deg_ref, h0_ref, sn_ref, dn_ref, hs_ref):
    d = deg_ref[...]
    dsrc = d[0, 0, :, 0:1] + d[1, 0, :, 0:1]
    ddst = d[0, 1, :, 0:1] + d[1, 1, :, 0:1]
    sn = lax.rsqrt(jnp.maximum(dsrc, 1.0))
    dn = lax.rsqrt(jnp.maximum(ddst, 1.0))
    sn_ref[...] = sn
    dn_ref[...] = dn
    hs_ref[...] = h0_ref[...] * sn


def _norm(degs, h0):
    return pl.pallas_call(
        _norm_body,
        grid=(N // _BN,),
        in_specs=[
            pl.BlockSpec((NC, 2, _BN, LANES), lambda i: (0, 0, i, 0)),
            pl.BlockSpec((_BN, N_CLASSES), lambda i: (i, 0)),
        ],
        out_specs=[
            pl.BlockSpec((_BN, 1), lambda i: (i, 0)),
            pl.BlockSpec((_BN, 1), lambda i: (i, 0)),
            pl.BlockSpec((_BN, N_CLASSES), lambda i: (i, 0)),
        ],
        out_shape=[
            jax.ShapeDtypeStruct((N, 1), jnp.float32),
            jax.ShapeDtypeStruct((N, 1), jnp.float32),
            jax.ShapeDtypeStruct((N, N_CLASSES), jnp.float32),
        ],
    )(degs, h0)


def _blend_body(agg_ref, dn_ref, sn_ref, h0_ref, h_ref, hs_ref):
    a = agg_ref[...]
    h = (1.0 - ALPHA) * (a[0] + a[1]) * dn_ref[...] + ALPHA * h0_ref[...]
    h_ref[...] = h
    hs_ref[...] = h * sn_ref[...]


def _blend(aggs, dn, sn, h0):
    return pl.pallas_call(
        _blend_body,
        grid=(N // _BN,),
        in_specs=[
            pl.BlockSpec((NC, _BN, N_CLASSES), lambda i: (0, i, 0)),
            pl.BlockSpec((_BN, 1), lambda i: (i, 0)),
            pl.BlockSpec((_BN, 1), lambda i: (i, 0)),
            pl.BlockSpec((_BN, N_CLASSES), lambda i: (i, 0)),
        ],
        out_specs=[
            pl.BlockSpec((_BN, N_CLASSES), lambda i: (i, 0)),
            pl.BlockSpec((_BN, N_CLASSES), lambda i: (i, 0)),
        ],
        out_shape=[
            jax.ShapeDtypeStruct((N, N_CLASSES), jnp.float32),
            jax.ShapeDtypeStruct((N, N_CLASSES), jnp.float32),
        ],
    )(aggs, dn, sn, h0)


def kernel(features, edge_index, W1, b1, W2, b2):
    src = edge_index[0]
    dst = edge_index[1]
    pad = EPAD - E
    src_gath = jnp.concatenate([src, jnp.zeros((pad,), jnp.int32)])
    src_deg = jnp.concatenate([src, jnp.full((pad,), N, jnp.int32)])
    dst_pad = jnp.concatenate([dst, jnp.full((pad,), N, jnp.int32)])

    h0 = _mlp(features, W1, b1.reshape(1, -1), W2, b2.reshape(1, -1))
    degs = _sc_degrees(src_deg, dst_pad)
    sn, dn, hs = _norm(degs, h0)
    h = h0
    for _ in range(K):
        aggs = _sc_step(hs, src_gath, dst_pad)
        h, hs = _blend(aggs, dn, sn, h0)
    return h


# speedup vs baseline: 4.8843x; 1.3867x over previous
import functools

import jax
import jax.numpy as jnp
from jax import lax
from jax.experimental import pallas as pl
from jax.experimental.pallas import tpu as pltpu
from jax.experimental.pallas import tpu_sc as plsc

N = 10000
E = 160000
D_IN = 256
H_FEATS = 512
N_CLASSES = 64
K = 10
ALPHA = 0.1

NC = 2
NS = 16
LANES = 16
NPAD = 10240
EPT = 5120
EPAD = NC * NS * EPT
CHUNK = 128
NCHUNK = EPT // CHUNK
NBUF = 8
NROUND = NCHUNK // NBUF
DEGSKEW = 4
ROWS_PER_TILE = NPAD // NS
_MESH = plsc.VectorSubcoreMesh(core_axis_name="c", subcore_axis_name="s")
_SC_PARAMS = pltpu.CompilerParams(use_tc_tiling_on_sc=False)

_BN = 1000


def _mlp_body(x_ref, w1_ref, b1_ref, w2_ref, b2_ref, o_ref):
    h = jnp.dot(x_ref[...], w1_ref[...], preferred_element_type=jnp.float32)
    h = jnp.maximum(h + b1_ref[...], 0.0)
    o_ref[...] = jnp.dot(h, w2_ref[...], preferred_element_type=jnp.float32) + b2_ref[...]


def _mlp(x, w1, b1, w2, b2):
    return pl.pallas_call(
        _mlp_body,
        grid=(N // _BN,),
        in_specs=[
            pl.BlockSpec((_BN, D_IN), lambda i: (i, 0)),
            pl.BlockSpec((D_IN, H_FEATS), lambda i: (0, 0)),
            pl.BlockSpec((1, H_FEATS), lambda i: (0, 0)),
            pl.BlockSpec((H_FEATS, N_CLASSES), lambda i: (0, 0)),
            pl.BlockSpec((1, N_CLASSES), lambda i: (0, 0)),
        ],
        out_specs=pl.BlockSpec((_BN, N_CLASSES), lambda i: (i, 0)),
        out_shape=jax.ShapeDtypeStruct((N, N_CLASSES), jnp.float32),
    )(x, w1, b1, w2, b2)


@functools.partial(
    pl.kernel,
    out_type=jax.ShapeDtypeStruct((NC, 2, NPAD, LANES), jnp.float32),
    mesh=_MESH,
    compiler_params=_SC_PARAMS,
    scratch_types=[
        pltpu.VMEM((NCHUNK, CHUNK), jnp.int32),
        pltpu.VMEM((NCHUNK, CHUNK), jnp.int32),
        pltpu.VMEM((CHUNK, LANES), jnp.float32),
        pltpu.VMEM((CHUNK, LANES), jnp.float32),
        pltpu.VMEM_SHARED((NPAD, LANES), jnp.float32),
        pltpu.VMEM_SHARED((NPAD, LANES), jnp.float32),
        pltpu.SemaphoreType.DMA,
        pltpu.SemaphoreType.DMA,
    ],
)
def _sc_degrees(src_hbm, dst_hbm, out_hbm, sidx_v, didx_v, ones_v, zero_v,
                dsrc_sh, ddst_sh, sem_s, sem_d):
    cid = lax.axis_index("c")
    sid = lax.axis_index("s")
    wid = cid * NS + sid

    @pl.loop(0, CHUNK)
    def _(r):
        ones_v[r, :] = jnp.full((LANES,), 1.0, jnp.float32)
        zero_v[r, :] = jnp.zeros((LANES,), jnp.float32)

    pltpu.sync_copy(src_hbm.at[wid], sidx_v)
    pltpu.sync_copy(dst_hbm.at[wid], didx_v)

    base = sid * ROWS_PER_TILE
    for b in range(ROWS_PER_TILE // CHUNK):
        pltpu.sync_copy(zero_v, dsrc_sh.at[pl.ds(base + b * CHUNK, CHUNK)])
        pltpu.sync_copy(zero_v, ddst_sh.at[pl.ds(base + b * CHUNK, CHUNK)])
    plsc.subcore_barrier()

    def _issue(c):
        pltpu.async_copy(ones_v, dsrc_sh.at[sidx_v.at[c]], sem_s, add=True)
        pltpu.async_copy(ones_v, ddst_sh.at[didx_v.at[c]], sem_d, add=True)

    def _drain(c):
        pltpu.make_async_copy(ones_v, dsrc_sh.at[sidx_v.at[c]], sem_s).wait()
        pltpu.make_async_copy(ones_v, ddst_sh.at[didx_v.at[c]], sem_d).wait()

    for c in range(DEGSKEW):
        _issue(c)

    @pl.loop(0, NCHUNK - DEGSKEW)
    def _(r):
        _issue(r + DEGSKEW)
        _drain(r)

    for c in range(NCHUNK - DEGSKEW, NCHUNK):
        _drain(c)

    plsc.subcore_barrier()
    pltpu.sync_copy(dsrc_sh.at[pl.ds(base, ROWS_PER_TILE)],
                    out_hbm.at[cid, 0, pl.ds(base, ROWS_PER_TILE)])
    pltpu.sync_copy(ddst_sh.at[pl.ds(base, ROWS_PER_TILE)],
                    out_hbm.at[cid, 1, pl.ds(base, ROWS_PER_TILE)])


@functools.partial(
    pl.kernel,
    out_type=jax.ShapeDtypeStruct((NC, NPAD, N_CLASSES), jnp.float32),
    mesh=_MESH,
    compiler_params=_SC_PARAMS,
    scratch_types=[
        pltpu.VMEM((NCHUNK, CHUNK), jnp.int32),
        pltpu.VMEM((NCHUNK, CHUNK), jnp.int32),
        pltpu.VMEM((NBUF, CHUNK, N_CLASSES), jnp.float32),
        pltpu.VMEM((CHUNK, N_CLASSES), jnp.float32),
        pltpu.VMEM_SHARED((NPAD, N_CLASSES), jnp.float32),
        pltpu.SemaphoreType.DMA((NBUF,)),
        pltpu.SemaphoreType.DMA((NBUF,)),
    ],
)
def _sc_step(hs_hbm, src_hbm, dst_hbm, out_hbm, sidx_v, didx_v, rows_v, zero_v,
             agg_sh, gsem, ssem):
    cid = lax.axis_index("c")
    sid = lax.axis_index("s")
    wid = cid * NS + sid

    @pl.loop(0, CHUNK)
    def _(r):
        @pl.loop(0, N_CLASSES, step=LANES)
        def _(j):
            zero_v[r, pl.ds(j, LANES)] = jnp.zeros((LANES,), jnp.float32)

    pltpu.sync_copy(src_hbm.at[wid], sidx_v)
    pltpu.sync_copy(dst_hbm.at[wid], didx_v)

    base = sid * ROWS_PER_TILE
    for b in range(ROWS_PER_TILE // CHUNK):
        pltpu.sync_copy(zero_v, agg_sh.at[pl.ds(base + b * CHUNK, CHUNK)])
    plsc.subcore_barrier()

    def _gather(c, b):
        pltpu.async_copy(hs_hbm.at[sidx_v.at[c]], rows_v.at[b], gsem.at[b])

    def _gather_wait(c, b):
        pltpu.make_async_copy(hs_hbm.at[sidx_v.at[c]], rows_v.at[b], gsem.at[b]).wait()

    def _scatter(c, b):
        pltpu.async_copy(rows_v.at[b], agg_sh.at[didx_v.at[c]], ssem.at[b], add=True)

    def _scatter_wait(c, b):
        pltpu.make_async_copy(rows_v.at[b], agg_sh.at[didx_v.at[c]], ssem.at[b]).wait()

    for b in range(NBUF):
        _gather(b, b)

    @pl.loop(0, NROUND - 1)
    def _(r):
        c0 = r * NBUF
        for b in range(NBUF):
            _gather_wait(c0 + b, b)
            _scatter(c0 + b, b)
        for b in range(NBUF):
            _scatter_wait(c0 + b, b)
            _gather(c0 + NBUF + b, b)

    c0 = (NROUND - 1) * NBUF
    for b in range(NBUF):
        _gather_wait(c0 + b, b)
        _scatter(c0 + b, b)
    for b in range(NBUF):
        _scatter_wait(c0 + b, b)

    plsc.subcore_barrier()
    pltpu.sync_copy(agg_sh.at[pl.ds(base, ROWS_PER_TILE)],
                    out_hbm.at[cid, pl.ds(base, ROWS_PER_TILE)])


def _norm_body(deg_ref, h0_ref, sn_ref, dn_ref, hs_ref):
    d = deg_ref[...]
    dsrc = d[0, 0, :, 0:1] + d[1, 0, :, 0:1]
    ddst = d[0, 1, :, 0:1] + d[1, 1, :, 0:1]
    sn = lax.rsqrt(jnp.maximum(dsrc, 1.0))
    dn = lax.rsqrt(jnp.maximum(ddst, 1.0))
    sn_ref[...] = sn
    dn_ref[...] = dn
    hs_ref[...] = h0_ref[...] * sn


def _norm(degs, h0):
    return pl.pallas_call(
        _norm_body,
        grid=(N // _BN,),
        in_specs=[
            pl.BlockSpec((NC, 2, _BN, LANES), lambda i: (0, 0, i, 0)),
            pl.BlockSpec((_BN, N_CLASSES), lambda i: (i, 0)),
        ],
        out_specs=[
            pl.BlockSpec((_BN, 1), lambda i: (i, 0)),
            pl.BlockSpec((_BN, 1), lambda i: (i, 0)),
            pl.BlockSpec((_BN, N_CLASSES), lambda i: (i, 0)),
        ],
        out_shape=[
            jax.ShapeDtypeStruct((N, 1), jnp.float32),
            jax.ShapeDtypeStruct((N, 1), jnp.float32),
            jax.ShapeDtypeStruct((N, N_CLASSES), jnp.float32),
        ],
    )(degs, h0)


def _blend_body(agg_ref, dn_ref, sn_ref, h0_ref, h_ref, hs_ref):
    a = agg_ref[...]
    h = (1.0 - ALPHA) * (a[0] + a[1]) * dn_ref[...] + ALPHA * h0_ref[...]
    h_ref[...] = h
    hs_ref[...] = h * sn_ref[...]


def _blend(aggs, dn, sn, h0):
    return pl.pallas_call(
        _blend_body,
        grid=(N // _BN,),
        in_specs=[
            pl.BlockSpec((NC, _BN, N_CLASSES), lambda i: (0, i, 0)),
            pl.BlockSpec((_BN, 1), lambda i: (i, 0)),
            pl.BlockSpec((_BN, 1), lambda i: (i, 0)),
            pl.BlockSpec((_BN, N_CLASSES), lambda i: (i, 0)),
        ],
        out_specs=[
            pl.BlockSpec((_BN, N_CLASSES), lambda i: (i, 0)),
            pl.BlockSpec((_BN, N_CLASSES), lambda i: (i, 0)),
        ],
        out_shape=[
            jax.ShapeDtypeStruct((N, N_CLASSES), jnp.float32),
            jax.ShapeDtypeStruct((N, N_CLASSES), jnp.float32),
        ],
    )(aggs, dn, sn, h0)


def kernel(features, edge_index, W1, b1, W2, b2):
    src = edge_index[0]
    dst = edge_index[1]
    pad = EPAD - E
    src_gath = jnp.concatenate([src, jnp.zeros((pad,), jnp.int32)])
    src_deg = jnp.concatenate([src, jnp.full((pad,), N, jnp.int32)])
    dst_pad = jnp.concatenate([dst, jnp.full((pad,), N, jnp.int32)])
    src_gath = src_gath.reshape(NC * NS, NCHUNK, CHUNK)
    src_deg = src_deg.reshape(NC * NS, NCHUNK, CHUNK)
    dst_pad = dst_pad.reshape(NC * NS, NCHUNK, CHUNK)

    h0 = _mlp(features, W1, b1.reshape(1, -1), W2, b2.reshape(1, -1))
    degs = _sc_degrees(src_deg, dst_pad)
    sn, dn, hs = _norm(degs, h0)
    h = h0
    for _ in range(K):
        aggs = _sc_step(hs, src_gath, dst_pad)
        h, hs = _blend(aggs, dn, sn, h0)
    return h
